# Initial kernel scaffold; baseline (speedup 1.0000x reference)
import jax
import jax.numpy as jnp
from jax.experimental import pallas as pl


def kernel(x, resolution, origin):
    raise NotImplementedError("write your pallas kernel here")


# speedup vs baseline: 1.0007x; 1.0007x over previous
import functools

import jax
import jax.numpy as jnp
from jax import lax
from jax.experimental import pallas as pl
from jax.experimental.pallas import tpu as pltpu
from jax.experimental.pallas import tpu_sc as plsc

_B, _T, _N = 16, 50, 50
_P = _N // 2
_H = _W = 64
_BT = _B * _T
_BLOCK = _H * _W * _P
_TOTAL = _BT * _BLOCK
_NC, _NS, _L = 2, 16, 16
_NW = _NC * _NS
_BT_PER = _BT // _NW
_REGION = _BT_PER * _BLOCK
_CHUNK = 64000
_NCHUNK = _REGION // _CHUNK
_WAVE = 10
_NGROUP = _BT_PER * 2
_IROWS = 7


def _raster_body(x_hbm, res_hbm, org_hbm, zsrc_hbm, ones_hbm, out_hbm,
                 xbuf, resbuf, orgbuf, zbuf, onesbuf, ipos, sem):
    wid = lax.axis_index("s") * _NC + lax.axis_index("c")
    base = wid * _REGION

    pltpu.sync_copy(x_hbm, xbuf)
    pltpu.sync_copy(res_hbm, resbuf)
    pltpu.sync_copy(org_hbm, orgbuf)
    pltpu.sync_copy(zsrc_hbm, zbuf)
    pltpu.sync_copy(ones_hbm, onesbuf)

    lane = lax.iota(jnp.int32, 16)
    last = None
    for btl in range(_BT_PER):
        bt = wid * _BT_PER + btl
        for g in range(2):
            p = jnp.minimum(lane + g * 16, _P - 1)
            cx = plsc.load_gather(xbuf, [bt * _N + 2 * p])
            cy = plsc.load_gather(xbuf, [bt * _N + 2 * p + 1])
            bidx = jnp.zeros((16,), jnp.int32) + 2 * bt
            r0 = plsc.load_gather(resbuf, [bidx])
            r1 = plsc.load_gather(resbuf, [bidx + 1])
            o0 = plsc.load_gather(orgbuf, [bidx])
            o1 = plsc.load_gather(orgbuf, [bidx + 1])
            row = jnp.clip((cy / r0 + o0).astype(jnp.int32), 0, _H - 1)
            col = jnp.clip((cx / r1 + o1).astype(jnp.int32), 0, _W - 1)
            gpos = (base + btl * _BLOCK + row * (_W * _P) + col * _P + p)
            off = (btl * 2 + g) * 16
            ipos[off // 128, pl.ds(off % 128, 16)] = gpos
            last = gpos
    for off in range(_NGROUP * 16, _IROWS * 128, 16):
        ipos[off // 128, pl.ds(off % 128, 16)] = last

    for w in range(0, _NCHUNK, _WAVE):
        copies = [
            pltpu.async_copy(
                zbuf, out_hbm.at[pl.ds(base + (w + i) * _CHUNK, _CHUNK)], sem)
            for i in range(_WAVE)
        ]
        for c in copies:
            c.wait()

    scat = [
        pltpu.async_copy(onesbuf, out_hbm.at[ipos.at[k]], sem)
        for k in range(_IROWS)
    ]
    for c in scat:
        c.wait()


_raster_sc = functools.partial(
    pl.kernel,
    out_type=jax.ShapeDtypeStruct((_TOTAL,), jnp.float32),
    mesh=plsc.VectorSubcoreMesh(core_axis_name="c", subcore_axis_name="s"),
    compiler_params=pltpu.CompilerParams(needs_layout_passes=False),
    scratch_types=[
        pltpu.VMEM((_B * _T * _N,), jnp.float32),
        pltpu.VMEM((_BT * 2,), jnp.float32),
        pltpu.VMEM((_BT * 2,), jnp.float32),
        pltpu.VMEM((_CHUNK,), jnp.float32),
        pltpu.VMEM((128,), jnp.float32),
        pltpu.VMEM((_IROWS, 128), jnp.int32),
        pltpu.SemaphoreType.DMA,
    ],
)(_raster_body)


def kernel(x, resolution, origin):
    zsrc = jnp.zeros((_CHUNK,), jnp.float32)
    ones = jnp.ones((128,), jnp.float32)
    flat = _raster_sc(x.reshape(-1), resolution.reshape(-1),
                      origin.reshape(-1), zsrc, ones)
    return flat.reshape(_B, _T, _H, _W, _P)


# speedup vs baseline: 12.5356x; 12.5271x over previous
import functools

import jax
import jax.numpy as jnp
from jax import lax
from jax.experimental import pallas as pl
from jax.experimental.pallas import tpu as pltpu
from jax.experimental.pallas import tpu_sc as plsc

_B, _T, _N = 16, 50, 50
_P = _N // 2
_H = _W = 64
_BT = _B * _T
_R = _BT * _P
_NC, _NS, _L = 2, 16, 16
_NW = _NC * _NS
_RPW = _R // _NW
_SLAB = 5
_NSLAB = _RPW // _SLAB
_NGROUP = 40


def _raster_body(x_hbm, res_hbm, org_hbm, zsrc_hbm, out_hbm,
                 xbuf, resbuf, orgbuf, slab, rowbuf, colbuf):
    wid = lax.axis_index("s") * _NC + lax.axis_index("c")

    pltpu.sync_copy(x_hbm, xbuf)
    pltpu.sync_copy(res_hbm, resbuf)
    pltpu.sync_copy(org_hbm, orgbuf)
    pltpu.sync_copy(zsrc_hbm, slab)

    lane = lax.iota(jnp.int32, 16)
    for i in range(_NGROUP):
        k = jnp.minimum(lane + i * 16, _RPW - 1)
        btl = k // _P
        p = k - btl * _P
        cx = plsc.load_gather(xbuf, [(wid * _P + btl) * _N + 2 * p])
        cy = plsc.load_gather(xbuf, [(wid * _P + btl) * _N + 2 * p + 1])
        bidx = (wid * _P + btl) * 2
        r0 = plsc.load_gather(resbuf, [bidx])
        r1 = plsc.load_gather(resbuf, [bidx + 1])
        o0 = plsc.load_gather(orgbuf, [bidx])
        o1 = plsc.load_gather(orgbuf, [bidx + 1])
        row = jnp.clip((cy / r0 + o0).astype(jnp.int32), 0, _H - 1)
        col = jnp.clip((cx / r1 + o1).astype(jnp.int32), 0, _W - 1)
        rowbuf[pl.ds(i * 16, 16)] = row
        colbuf[pl.ds(i * 16, 16)] = col

    ones = jnp.ones((16,), jnp.float32)
    zeros = jnp.zeros((16,), jnp.float32)
    jc = jnp.minimum(lane, _SLAB - 1)
    m = lane < _SLAB
    base = wid * _RPW

    def _slab(c, carry):
        rv = rowbuf[pl.ds(c * _SLAB, 16)]
        cv = colbuf[pl.ds(c * _SLAB, 16)]
        plsc.store_scatter(slab, [jc, rv, cv], ones, mask=m)
        pltpu.sync_copy(slab, out_hbm.at[pl.ds(base + c * _SLAB, _SLAB)])
        plsc.store_scatter(slab, [jc, rv, cv], zeros, mask=m)
        return carry
    lax.fori_loop(0, _NSLAB, _slab, 0)


_raster_sc = functools.partial(
    pl.kernel,
    out_type=jax.ShapeDtypeStruct((_R, _H, _W), jnp.float32),
    mesh=plsc.VectorSubcoreMesh(core_axis_name="c", subcore_axis_name="s"),
    compiler_params=pltpu.CompilerParams(needs_layout_passes=False),
    scratch_types=[
        pltpu.VMEM((_B * _T * _N,), jnp.float32),
        pltpu.VMEM((_BT * 2,), jnp.float32),
        pltpu.VMEM((_BT * 2,), jnp.float32),
        pltpu.VMEM((_SLAB, _H, _W), jnp.float32),
        pltpu.VMEM((_NGROUP * 16,), jnp.int32),
        pltpu.VMEM((_NGROUP * 16,), jnp.int32),
    ],
)(_raster_body)


def kernel(x, resolution, origin):
    zsrc = jnp.zeros((_SLAB, _H, _W), jnp.float32)
    planes = _raster_sc(x.reshape(-1), resolution.reshape(-1),
                        origin.reshape(-1), zsrc)
    return planes.reshape(_B, _T, _P, _H, _W).transpose(0, 1, 3, 4, 2)
